# baseline (speedup 1.0000x reference)
import functools

import jax
import jax.numpy as jnp
from jax import lax
from jax.experimental import pallas as pl
from jax.experimental.pallas import tpu as pltpu
from jax.experimental.pallas import tpu_sc as plsc

NC = 2
NS = 16
NW = NC * NS


def _sc_gather_t(xT, tT):
    P, B = xT.shape
    D, V = tT.shape
    L = 16

    mesh = plsc.VectorSubcoreMesh(core_axis_name="c", subcore_axis_name="s")

    @functools.partial(
        pl.kernel,
        out_type=jax.ShapeDtypeStruct((P, D, B), jnp.float32),
        mesh=mesh,
        compiler_params=pltpu.CompilerParams(
            use_tc_tiling_on_sc=True, needs_layout_passes=False
        ),
        scratch_types=[
            pltpu.VMEM((V,), jnp.float32),
            pltpu.VMEM((2, B), jnp.int32),
            pltpu.VMEM((2, B), jnp.float32),
            pltpu.SemaphoreType.DMA,
            pltpu.SemaphoreType.DMA,
            pltpu.SemaphoreType.DMA,
        ],
    )
    def k(xT_hbm, tT_hbm, out_hbm, trow, idxb, outb, tsem, isem, osem):
        f = lax.axis_index("s") * NC + lax.axis_index("c")
        row_cp = pltpu.async_copy(tT_hbm.at[f], trow, tsem)

        def fire_idx(p, slot):
            pltpu.async_copy(xT_hbm.at[p], idxb.at[slot], isem)

        def drain_idx(p, slot):
            pltpu.make_async_copy(xT_hbm.at[p], idxb.at[slot], isem).wait()

        def fire_out(p, slot):
            pltpu.async_copy(outb.at[slot], out_hbm.at[p, f], osem)

        def drain_out(p, slot):
            pltpu.make_async_copy(outb.at[slot], out_hbm.at[p, f], osem).wait()

        fire_idx(0, 0)
        row_cp.wait()

        def compute(islot, oslot):
            @pl.loop(0, B // L, unroll=8)
            def _(i):
                idx = idxb[islot, pl.ds(i * L, L)]
                outb[oslot, pl.ds(i * L, L)] = plsc.load_gather(trow, [idx])

        @pl.loop(0, P, step=2)
        def _(p2):
            fire_idx(p2 + 1, 1)
            drain_idx(p2, 0)
            compute(0, 0)

            @pl.when(p2 > 0)
            def _():
                drain_out(p2 - 1, 1)

            fire_out(p2, 0)

            @pl.when(p2 < P - 2)
            def _():
                fire_idx(p2 + 2, 0)

            drain_idx(p2 + 1, 1)
            compute(1, 1)
            drain_out(p2, 0)
            fire_out(p2 + 1, 1)

        drain_out(P - 1, 1)

    return k(xT, tT)


def kernel(x, table):
    B0, B1 = x.shape
    D = table.shape[1]
    outT = _sc_gather_t(x.T, table.T)
    return outT.transpose(2, 0, 1)


# speedup vs baseline: 1.3910x; 1.3910x over previous
import functools

import jax
import jax.numpy as jnp
from jax import lax
from jax.experimental import pallas as pl
from jax.experimental.pallas import tpu as pltpu
from jax.experimental.pallas import tpu_sc as plsc

NC = 2
NS = 16
NW = NC * NS


def _sc_gather_t(xT, tT):
    P, B = xT.shape
    D, V = tT.shape
    L = 16

    mesh = plsc.VectorSubcoreMesh(core_axis_name="c", subcore_axis_name="s")

    @functools.partial(
        pl.kernel,
        out_type=jax.ShapeDtypeStruct((P, D, B), jnp.float32),
        mesh=mesh,
        compiler_params=pltpu.CompilerParams(
            use_tc_tiling_on_sc=True, needs_layout_passes=False
        ),
        scratch_types=[
            pltpu.VMEM((V,), jnp.float32),
            pltpu.VMEM((2, B), jnp.int32),
            pltpu.VMEM((2, B), jnp.float32),
            pltpu.SemaphoreType.DMA,
            pltpu.SemaphoreType.DMA,
            pltpu.SemaphoreType.DMA,
        ],
    )
    def k(xT_hbm, tT_hbm, out_hbm, trow, idxb, outb, tsem, isem, osem):
        f = lax.axis_index("s") * NC + lax.axis_index("c")
        row_cp = pltpu.async_copy(tT_hbm.at[f], trow, tsem)

        def fire_idx(p, slot):
            pltpu.async_copy(xT_hbm.at[p], idxb.at[slot], isem)

        def drain_idx(p, slot):
            pltpu.make_async_copy(xT_hbm.at[p], idxb.at[slot], isem).wait()

        def fire_out(p, slot):
            pltpu.async_copy(outb.at[slot], out_hbm.at[p, f], osem)

        def drain_out(p, slot):
            pltpu.make_async_copy(outb.at[slot], out_hbm.at[p, f], osem).wait()

        fire_idx(0, 0)
        row_cp.wait()

        def compute(islot, oslot):
            @pl.loop(0, B // L)
            def _(i):
                idx = idxb[islot, pl.ds(i * L, L)]
                outb[oslot, pl.ds(i * L, L)] = plsc.load_gather(trow, [idx])

        @pl.loop(0, P, step=2)
        def _(p2):
            fire_idx(p2 + 1, 1)
            drain_idx(p2, 0)
            compute(0, 0)

            @pl.when(p2 > 0)
            def _():
                drain_out(p2 - 1, 1)

            fire_out(p2, 0)

            @pl.when(p2 < P - 2)
            def _():
                fire_idx(p2 + 2, 0)

            drain_idx(p2 + 1, 1)
            compute(1, 1)
            drain_out(p2, 0)
            fire_out(p2 + 1, 1)

        drain_out(P - 1, 1)

    return k(xT, tT)


def kernel(x, table):
    B0, B1 = x.shape
    D = table.shape[1]
    outT = _sc_gather_t(x.T, table.T)
    return outT.transpose(2, 0, 1)


# speedup vs baseline: 2.6627x; 1.9142x over previous
import functools

import jax
import jax.numpy as jnp
from jax import lax
from jax.experimental import pallas as pl
from jax.experimental.pallas import tpu as pltpu
from jax.experimental.pallas import tpu_sc as plsc

NC = 2
NS = 16
NW = NC * NS


def _sc_gather_t(xT, tT):
    P, B = xT.shape
    D, V = tT.shape
    L = 16

    mesh = plsc.VectorSubcoreMesh(core_axis_name="c", subcore_axis_name="s")

    @functools.partial(
        pl.kernel,
        out_type=jax.ShapeDtypeStruct((P, D, B), jnp.float32),
        mesh=mesh,
        compiler_params=pltpu.CompilerParams(
            use_tc_tiling_on_sc=True, needs_layout_passes=False
        ),
        scratch_types=[
            pltpu.VMEM((V,), jnp.float32),
            pltpu.VMEM((2, B), jnp.int32),
            pltpu.VMEM((2, B), jnp.float32),
            pltpu.SemaphoreType.DMA,
            pltpu.SemaphoreType.DMA,
            pltpu.SemaphoreType.DMA,
        ],
    )
    def k(xT_hbm, tT_hbm, out_hbm, trow, idxb, outb, tsem, isem, osem):
        f = lax.axis_index("s") * NC + lax.axis_index("c")
        row_cp = pltpu.async_copy(tT_hbm.at[f], trow, tsem)

        def fire_idx(p, slot):
            pltpu.async_copy(xT_hbm.at[p], idxb.at[slot], isem)

        def drain_idx(p, slot):
            pltpu.make_async_copy(xT_hbm.at[p], idxb.at[slot], isem).wait()

        def fire_out(p, slot):
            pltpu.async_copy(outb.at[slot], out_hbm.at[p, f], osem)

        def drain_out(p, slot):
            pltpu.make_async_copy(outb.at[slot], out_hbm.at[p, f], osem).wait()

        fire_idx(0, 0)
        row_cp.wait()

        def compute(islot, oslot):
            @plsc.parallel_loop(0, B, step=L, unroll=4)
            def _(i):
                idx = idxb[islot, pl.ds(i, L)]
                outb[oslot, pl.ds(i, L)] = plsc.load_gather(trow, [idx])

        @pl.loop(0, P, step=2)
        def _(p2):
            fire_idx(p2 + 1, 1)
            drain_idx(p2, 0)
            compute(0, 0)

            @pl.when(p2 > 0)
            def _():
                drain_out(p2 - 1, 1)

            fire_out(p2, 0)

            @pl.when(p2 < P - 2)
            def _():
                fire_idx(p2 + 2, 0)

            drain_idx(p2 + 1, 1)
            compute(1, 1)
            drain_out(p2, 0)
            fire_out(p2 + 1, 1)

        drain_out(P - 1, 1)

    return k(xT, tT)


def kernel(x, table):
    B0, B1 = x.shape
    D = table.shape[1]
    outT = _sc_gather_t(x.T, table.T)
    return outT.transpose(2, 0, 1)


# speedup vs baseline: 2.6858x; 1.0087x over previous
import functools

import jax
import jax.numpy as jnp
from jax import lax
from jax.experimental import pallas as pl
from jax.experimental.pallas import tpu as pltpu
from jax.experimental.pallas import tpu_sc as plsc

NC = 2
NS = 16
NW = NC * NS


def _sc_gather_t(xT, tT):
    P, B = xT.shape
    D, V = tT.shape
    L = 16

    mesh = plsc.VectorSubcoreMesh(core_axis_name="c", subcore_axis_name="s")

    @functools.partial(
        pl.kernel,
        out_type=jax.ShapeDtypeStruct((P, D, B), jnp.float32),
        mesh=mesh,
        compiler_params=pltpu.CompilerParams(
            use_tc_tiling_on_sc=True, needs_layout_passes=False
        ),
        scratch_types=[
            pltpu.VMEM((V,), jnp.float32),
            pltpu.VMEM((2, B), jnp.int32),
            pltpu.VMEM((2, B), jnp.float32),
            pltpu.SemaphoreType.DMA,
            pltpu.SemaphoreType.DMA,
            pltpu.SemaphoreType.DMA,
        ],
    )
    def k(xT_hbm, tT_hbm, out_hbm, trow, idxb, outb, tsem, isem, osem):
        f = lax.axis_index("s") * NC + lax.axis_index("c")
        row_cp = pltpu.async_copy(tT_hbm.at[f], trow, tsem)

        def fire_idx(p, slot):
            pltpu.async_copy(xT_hbm.at[p], idxb.at[slot], isem)

        def drain_idx(p, slot):
            pltpu.make_async_copy(xT_hbm.at[p], idxb.at[slot], isem).wait()

        def fire_out(p, slot):
            pltpu.async_copy(outb.at[slot], out_hbm.at[p, f], osem)

        def drain_out(p, slot):
            pltpu.make_async_copy(outb.at[slot], out_hbm.at[p, f], osem).wait()

        fire_idx(0, 0)
        row_cp.wait()

        def compute(islot, oslot):
            @plsc.parallel_loop(0, B, step=L, unroll=8)
            def _(i):
                idx = idxb[islot, pl.ds(i, L)]
                outb[oslot, pl.ds(i, L)] = plsc.load_gather(trow, [idx])

        @pl.loop(0, P, step=2)
        def _(p2):
            fire_idx(p2 + 1, 1)
            drain_idx(p2, 0)
            compute(0, 0)

            @pl.when(p2 > 0)
            def _():
                drain_out(p2 - 1, 1)

            fire_out(p2, 0)

            @pl.when(p2 < P - 2)
            def _():
                fire_idx(p2 + 2, 0)

            drain_idx(p2 + 1, 1)
            compute(1, 1)
            drain_out(p2, 0)
            fire_out(p2 + 1, 1)

        drain_out(P - 1, 1)

    return k(xT, tT)


def kernel(x, table):
    B0, B1 = x.shape
    D = table.shape[1]
    outT = _sc_gather_t(x.T, table.T)
    return outT.transpose(2, 0, 1)


# speedup vs baseline: 3.0340x; 1.1296x over previous
import functools

import jax
import jax.numpy as jnp
from jax import lax
from jax.experimental import pallas as pl
from jax.experimental.pallas import tpu as pltpu
from jax.experimental.pallas import tpu_sc as plsc

NC = 2
NS = 16
NW = NC * NS


def _sc_gather_t(xT, tT):
    P, B = xT.shape
    D, V = tT.shape
    L = 16

    mesh = plsc.VectorSubcoreMesh(core_axis_name="c", subcore_axis_name="s")

    @functools.partial(
        pl.kernel,
        out_type=jax.ShapeDtypeStruct((P, D, B), jnp.float32),
        mesh=mesh,
        compiler_params=pltpu.CompilerParams(
            use_tc_tiling_on_sc=True, needs_layout_passes=False
        ),
        scratch_types=[
            pltpu.VMEM((V,), jnp.float32),
            pltpu.VMEM((4, B), jnp.int32),
            pltpu.VMEM((2, B), jnp.float32),
            pltpu.SemaphoreType.DMA,
            pltpu.SemaphoreType.DMA,
            pltpu.SemaphoreType.DMA,
            pltpu.SemaphoreType.DMA,
            pltpu.SemaphoreType.DMA,
            pltpu.SemaphoreType.DMA,
            pltpu.SemaphoreType.DMA,
        ],
    )
    def k(xT_hbm, tT_hbm, out_hbm, trow, idxb, outb, tsem,
          i0, i1, i2, i3, o0, o1):
        f = lax.axis_index("s") * NC + lax.axis_index("c")
        row_cp = pltpu.async_copy(tT_hbm.at[f], trow, tsem)
        isems = (i0, i1, i2, i3)
        osems = (o0, o1)

        def fire_idx(p, slot):
            pltpu.async_copy(xT_hbm.at[p], idxb.at[slot], isems[slot])

        def drain_idx(p, slot):
            pltpu.make_async_copy(xT_hbm.at[p], idxb.at[slot], isems[slot]).wait()

        def fire_out(p, slot):
            pltpu.async_copy(outb.at[slot], out_hbm.at[p, f], osems[slot])

        def drain_out(p, slot):
            pltpu.make_async_copy(outb.at[slot], out_hbm.at[p, f], osems[slot]).wait()

        for b in range(3):
            fire_idx(b, b)
        row_cp.wait()

        def compute(islot, oslot):
            @plsc.parallel_loop(0, B, step=L, unroll=8)
            def _(i):
                idx = idxb[islot, pl.ds(i, L)]
                outb[oslot, pl.ds(i, L)] = plsc.load_gather(trow, [idx])

        @pl.loop(0, P, step=4)
        def _(p4):
            for b in range(4):
                t = p4 + b

                @pl.when(t + 3 < P)
                def _():
                    fire_idx(t + 3, (b + 3) % 4)

                @pl.when(t >= 2)
                def _():
                    drain_out(t - 2, b % 2)

                drain_idx(t, b)
                compute(b, b % 2)
                fire_out(t, b % 2)

        drain_out(P - 2, 0)
        drain_out(P - 1, 1)

    return k(xT, tT)


def kernel(x, table):
    B0, B1 = x.shape
    D = table.shape[1]
    outT = _sc_gather_t(x.T, table.T)
    return outT.transpose(2, 0, 1)
